# baseline (speedup 1.0000x reference)
import functools

import jax
import jax.numpy as jnp
from jax.experimental import pallas as pl
from jax.experimental.pallas import tpu as pltpu


def _nnue_body(nf, ft, wf_ref, bf_ref, turn_ref, l0w_ref, l0b_ref, l1w_ref,
               l1b_ref, l2w_ref, l2b_ref, out_ref, acc_ref):
    j = pl.program_id(1)

    @pl.when(j == 0)
    def _init():
        acc_ref[...] = jnp.zeros_like(acc_ref)

    w0 = l0w_ref[:, pl.ds(j * ft, ft)]
    pw = jax.lax.dot_general(wf_ref[...], w0, (((1,), (1,)), ((), ())),
                             preferred_element_type=jnp.float32)
    pb = jax.lax.dot_general(bf_ref[...], w0, (((1,), (1,)), ((), ())),
                             preferred_element_type=jnp.float32)
    acc_ref[...] += jnp.concatenate([pw, pb], axis=1)

    @pl.when(j == nf - 1)
    def _epilogue():
        acc = acc_ref[...]
        m = acc.shape[1] // 2
        w = acc[:, :m] + l0b_ref[...]
        b = acc[:, m:] + l0b_ref[...]
        t = turn_ref[...]
        a = t * jnp.concatenate([w, b], axis=1) \
            + (1.0 - t) * jnp.concatenate([b, w], axis=1)
        l1_x = jnp.clip(a, 0.0, 1.0)
        h = jax.lax.dot_general(l1_x, l1w_ref[...], (((1,), (1,)), ((), ())),
                                preferred_element_type=jnp.float32) + l1b_ref[...]
        l2_x = jnp.clip(h, 0.0, 1.0)
        out_ref[...] = (jnp.sum(l2_x * l2w_ref[...], axis=1, keepdims=True)
                        + l2b_ref[0, 0])


def kernel(white_features, black_features, turn, score, result,
           l0_w, l0_b, l1_w, l1_b, l2_w, l2_b):
    del score, result
    B, F = white_features.shape
    M = l0_w.shape[0]
    N = l1_w.shape[0]
    K = l2_w.shape[0]

    bt = 1024 if B % 1024 == 0 else B
    ft = 2560 if F % 2560 == 0 else F
    nb, nf = B // bt, F // ft

    turn_b = jnp.broadcast_to(turn, (B, 2 * M))
    l0_b2 = l0_b.reshape(1, M)
    l1_b2 = l1_b.reshape(1, N)
    l2_b2 = l2_b.reshape(1, K)

    body = functools.partial(_nnue_body, nf, ft)

    grid_spec = pltpu.PrefetchScalarGridSpec(
        num_scalar_prefetch=0,
        grid=(nb, nf),
        in_specs=[
            pl.BlockSpec((bt, ft), lambda i, j: (i, j)),
            pl.BlockSpec((bt, ft), lambda i, j: (i, j)),
            pl.BlockSpec((bt, 2 * M), lambda i, j: (i, 0)),
            pl.BlockSpec((M, F), lambda i, j: (0, 0)),
            pl.BlockSpec((1, M), lambda i, j: (0, 0)),
            pl.BlockSpec((N, 2 * M), lambda i, j: (0, 0)),
            pl.BlockSpec((1, N), lambda i, j: (0, 0)),
            pl.BlockSpec((K, N), lambda i, j: (0, 0)),
            pl.BlockSpec(memory_space=pltpu.SMEM),
        ],
        out_specs=pl.BlockSpec((bt, K), lambda i, j: (i, 0)),
        scratch_shapes=[pltpu.VMEM((bt, 2 * M), jnp.float32)],
    )

    return pl.pallas_call(
        body,
        grid_spec=grid_spec,
        out_shape=jax.ShapeDtypeStruct((B, K), jnp.float32),
        compiler_params=pltpu.CompilerParams(
            dimension_semantics=("parallel", "arbitrary"),
        ),
    )(white_features, black_features, turn_b, l0_w, l0_b2, l1_w, l1_b2,
      l2_w, l2_b2)


# speedup vs baseline: 1.0029x; 1.0029x over previous
import functools

import jax
import jax.numpy as jnp
from jax.experimental import pallas as pl
from jax.experimental.pallas import tpu as pltpu


def _nnue_body(nf, ft, wf_ref, bf_ref, turn_ref, l0w_ref, l0b_ref, l1w_ref,
               l1b_ref, l2w_ref, l2b_ref, out_ref, acc_ref):
    j = pl.program_id(1)

    @pl.when(j == 0)
    def _init():
        acc_ref[...] = jnp.zeros_like(acc_ref)

    w0 = l0w_ref[:, pl.ds(j * ft, ft)]
    pw = jax.lax.dot_general(wf_ref[...], w0, (((1,), (1,)), ((), ())),
                             preferred_element_type=jnp.float32)
    pb = jax.lax.dot_general(bf_ref[...], w0, (((1,), (1,)), ((), ())),
                             preferred_element_type=jnp.float32)
    acc_ref[...] += jnp.concatenate([pw, pb], axis=1)

    @pl.when(j == nf - 1)
    def _epilogue():
        acc = acc_ref[...]
        m = acc.shape[1] // 2
        w = acc[:, :m] + l0b_ref[...]
        b = acc[:, m:] + l0b_ref[...]
        t = turn_ref[...]
        a = t * jnp.concatenate([w, b], axis=1) \
            + (1.0 - t) * jnp.concatenate([b, w], axis=1)
        l1_x = jnp.clip(a, 0.0, 1.0)
        h = jax.lax.dot_general(l1_x, l1w_ref[...], (((1,), (1,)), ((), ())),
                                preferred_element_type=jnp.float32) + l1b_ref[...]
        l2_x = jnp.clip(h, 0.0, 1.0)
        out_ref[...] = (jnp.sum(l2_x * l2w_ref[...], axis=1, keepdims=True)
                        + l2b_ref[0, 0])


def kernel(white_features, black_features, turn, score, result,
           l0_w, l0_b, l1_w, l1_b, l2_w, l2_b):
    del score, result
    B, F = white_features.shape
    M = l0_w.shape[0]
    N = l1_w.shape[0]
    K = l2_w.shape[0]

    bt = 1024 if B % 1024 == 0 else B
    ft = 2048 if F % 2048 == 0 else F
    nb, nf = B // bt, F // ft

    turn_b = jnp.broadcast_to(turn, (B, 2 * M))
    l0_b2 = l0_b.reshape(1, M)
    l1_b2 = l1_b.reshape(1, N)
    l2_b2 = l2_b.reshape(1, K)

    body = functools.partial(_nnue_body, nf, ft)

    grid_spec = pltpu.PrefetchScalarGridSpec(
        num_scalar_prefetch=0,
        grid=(nb, nf),
        in_specs=[
            pl.BlockSpec((bt, ft), lambda i, j: (i, j)),
            pl.BlockSpec((bt, ft), lambda i, j: (i, j)),
            pl.BlockSpec((bt, 2 * M), lambda i, j: (i, 0)),
            pl.BlockSpec((M, F), lambda i, j: (0, 0)),
            pl.BlockSpec((1, M), lambda i, j: (0, 0)),
            pl.BlockSpec((N, 2 * M), lambda i, j: (0, 0)),
            pl.BlockSpec((1, N), lambda i, j: (0, 0)),
            pl.BlockSpec((K, N), lambda i, j: (0, 0)),
            pl.BlockSpec(memory_space=pltpu.SMEM),
        ],
        out_specs=pl.BlockSpec((bt, K), lambda i, j: (i, 0)),
        scratch_shapes=[pltpu.VMEM((bt, 2 * M), jnp.float32)],
    )

    return pl.pallas_call(
        body,
        grid_spec=grid_spec,
        out_shape=jax.ShapeDtypeStruct((B, K), jnp.float32),
        compiler_params=pltpu.CompilerParams(
            dimension_semantics=("parallel", "arbitrary"),
        ),
    )(white_features, black_features, turn_b, l0_w, l0_b2, l1_w, l1_b2,
      l2_w, l2_b2)
